# baseline (speedup 1.0000x reference)
import functools
import math

import jax
import jax.numpy as jnp
from jax import lax
from jax.experimental import pallas as pl
from jax.experimental.pallas import tpu as pltpu
from jax.experimental.pallas import tpu_sc as plsc

B, M, L, H, NH, D, E, TOPK, F = 1, 2048, 2048, 768, 12, 64, 8, 2, 3072
MU, GAMMA = 0.7, 1.0

BM = 256
BA = 1024
KB = L // BA + 1
CAP = 2 * M + E * BM
NBLK = CAP // BM
BF = 3072
NFB = F // BF
NW = 32
PAIRS = 2 * M
PPW = PAIRS // NW
SCALE = 1.0 / math.sqrt(float(H))


def _proj_body(x_ref, w_ref, *o_refs):
    r = jnp.dot(x_ref[...], w_ref[...], preferred_element_type=jnp.float32)
    for g, o_ref in enumerate(o_refs):
        for h in range(NH):
            o_ref[h] = r[:, (g * NH + h) * D:(g * NH + h + 1) * D]


def _proj_heads(x, w):
    n, k = x.shape
    k2, m = w.shape
    ng = m // H
    return pl.pallas_call(
        _proj_body,
        grid=(n // BM,),
        in_specs=[
            pl.BlockSpec((BM, k), lambda i: (i, 0)),
            pl.BlockSpec((k2, m), lambda i: (0, 0)),
        ],
        out_specs=[pl.BlockSpec((NH, BM, D), lambda i: (0, i, 0))] * ng,
        out_shape=[jax.ShapeDtypeStruct((NH, n, D), jnp.float32)] * ng,
    )(x, w)


def _flash_body(q_ref, k_ref, v_ref, pos_ref, o_ref, s_s, v_s, pz_s):
    kb = pl.program_id(2)
    q = q_ref[0]
    k = k_ref[0]
    s = lax.dot_general(q, k, (((1,), (1,)), ((), ())),
                        preferred_element_type=jnp.float32)
    ri = lax.broadcasted_iota(jnp.int32, (BA, BA), 0)
    cj = lax.broadcasted_iota(jnp.int32, (BA, BA), 1)

    @pl.when(kb == 0)
    def _():
        pz_s[...] = jnp.zeros_like(pz_s)

    pzb = jnp.dot(q, pos_ref[:, pl.ds((kb + 1) * BA, BA)],
                  preferred_element_type=jnp.float32)
    rotb = pltpu.roll(pzb, 0, 1, stride=1, stride_axis=0)
    pbias = jnp.where(cj >= ri, rotb, pz_s[...])
    pz_s[...] = rotb
    s = (s + pbias) * SCALE

    @pl.when((kb == 0) | (kb == KB - 1))
    def _():
        rel = kb * BA + cj - ri
        valid = (rel >= 0) & (rel < L)
        s_s[:, pl.ds(kb * BA, BA)] = jnp.where(valid, s, -1e30)

    @pl.when((kb != 0) & (kb != KB - 1))
    def _():
        s_s[:, pl.ds(kb * BA, BA)] = s

    v_s[pl.ds(kb * BA, BA), :] = v_ref[0]

    @pl.when(kb == KB - 1)
    def _():
        sall = s_s[...]
        mrow = jnp.max(sall, axis=1, keepdims=True)
        pm = jnp.exp(sall - mrow)
        p = pm * (1.0 / jnp.sum(pm, axis=1, keepdims=True))
        o_ref[0] = jnp.dot(p, v_s[...], preferred_element_type=jnp.float32)


def _flash(qh, kh, vh, pos_pad):
    return pl.pallas_call(
        _flash_body,
        grid=(NH, M // BA, KB),
        in_specs=[
            pl.BlockSpec((1, BA, D), lambda h, i, j: (h, i, 0)),
            pl.BlockSpec((1, BA, D), lambda h, i, j: (h, i + j, 0)),
            pl.BlockSpec((1, BA, D), lambda h, i, j: (h, i + j, 0)),
            pl.BlockSpec((D, L + 2 * BA), lambda h, i, j: (0, 0)),
        ],
        out_specs=pl.BlockSpec((1, BA, D), lambda h, i, j: (h, i, 0)),
        out_shape=jax.ShapeDtypeStruct((NH, M, D), jnp.float32),
        scratch_shapes=[
            pltpu.VMEM((BA, KB * BA), jnp.float32),
            pltpu.VMEM((KB * BA, D), jnp.float32),
            pltpu.VMEM((BA, BA), jnp.float32),
        ],
    )(qh, kh, vh, pos_pad)


def _post_body(a_ref, h_ref, wo_ref, g_ref, b_ref, wg_ref, h1_ref, r_ref):
    att = jnp.concatenate([a_ref[h] for h in range(NH)], axis=1)
    x = jnp.dot(att, wo_ref[...],
                preferred_element_type=jnp.float32) + h_ref[...]
    mu = jnp.mean(x, axis=1, keepdims=True)
    var = jnp.mean((x - mu) ** 2, axis=1, keepdims=True)
    xn = (x - mu) / jnp.sqrt(var + 1e-5) * g_ref[...] + b_ref[...]
    h1_ref[...] = xn

    logits = jnp.dot(xn, wg_ref[...], preferred_element_type=jnp.float32)
    eidx = lax.broadcasted_iota(jnp.int32, (BM, E), 1)
    v1 = jnp.max(logits, axis=1, keepdims=True)
    i1 = jnp.min(jnp.where(logits == v1, eidx, E), axis=1, keepdims=True)
    l2 = jnp.where(eidx == i1, -jnp.inf, logits)
    v2 = jnp.max(l2, axis=1, keepdims=True)
    i2 = jnp.min(jnp.where(l2 == v2, eidx, E), axis=1, keepdims=True)
    e2 = jnp.exp(v2 - v1)
    w1 = 1.0 / (1.0 + e2)
    w2 = e2 / (1.0 + e2)
    r = jnp.where(eidx == 0, i1.astype(jnp.float32), 0.0)
    r = r + jnp.where(eidx == 1, i2.astype(jnp.float32), 0.0)
    r = r + jnp.where(eidx == 2, w1, 0.0)
    r = r + jnp.where(eidx == 3, w2, 0.0)
    r_ref[...] = r


def _post_attn(att, h2d, wo, g, b, wg):
    return pl.pallas_call(
        _post_body,
        grid=(M // BM,),
        in_specs=[
            pl.BlockSpec((NH, BM, D), lambda i: (0, i, 0)),
            pl.BlockSpec((BM, H), lambda i: (i, 0)),
            pl.BlockSpec((H, H), lambda i: (0, 0)),
            pl.BlockSpec((1, H), lambda i: (0, 0)),
            pl.BlockSpec((1, H), lambda i: (0, 0)),
            pl.BlockSpec((H, E), lambda i: (0, 0)),
        ],
        out_specs=[
            pl.BlockSpec((BM, H), lambda i: (i, 0)),
            pl.BlockSpec((BM, E), lambda i: (i, 0)),
        ],
        out_shape=[
            jax.ShapeDtypeStruct((M, H), jnp.float32),
            jax.ShapeDtypeStruct((M, E), jnp.float32),
        ],
    )(att, h2d, wo, g, b, wg)


def _route_body(rfull_ref, rchunk_ref, tri_ref, ut_ref, dst_ref, be_ref):
    i = pl.program_id(0)
    rfull = rfull_ref[...]
    eidx_f = lax.broadcasted_iota(jnp.int32, (M, E), 1).astype(jnp.float32)
    oh1 = (eidx_f == rfull[:, 0:1]).astype(jnp.float32)
    oh2 = (eidx_f == rfull[:, 1:2]).astype(jnp.float32)
    counts1 = jnp.sum(oh1, axis=0, keepdims=True)
    counts = counts1 + jnp.sum(oh2, axis=0, keepdims=True)
    counts_i = counts.astype(jnp.int32)
    pc = ((counts_i + BM - 1) // BM) * BM
    pcf = jnp.broadcast_to(pc.astype(jnp.float32), (E, E))
    po8 = jnp.dot(pcf, ut_ref[...], preferred_element_type=jnp.float32)
    po = po8[0:1, :]

    tri = tri_ref[...]
    excl1 = jnp.dot(tri, oh1, preferred_element_type=jnp.float32)
    excl2 = jnp.dot(tri, oh2, preferred_element_type=jnp.float32)

    rchunk = rchunk_ref[...]
    eidx_c = lax.broadcasted_iota(jnp.int32, (BM, E), 1).astype(jnp.float32)
    oh1c = (eidx_c == rchunk[:, 0:1]).astype(jnp.float32)
    oh2c = (eidx_c == rchunk[:, 1:2]).astype(jnp.float32)
    rank1 = jnp.sum(excl1 * oh1c, axis=1, keepdims=True)
    rank2 = jnp.sum(excl2 * oh2c, axis=1, keepdims=True) + \
        jnp.sum(oh1c * 0.0 + oh2c * counts1, axis=1, keepdims=True)
    dst1 = jnp.sum(oh1c * po, axis=1, keepdims=True) + rank1
    dst2 = jnp.sum(oh2c * po, axis=1, keepdims=True) + rank2
    dst_ref[...] = jnp.concatenate(
        [dst1.astype(jnp.int32), dst2.astype(jnp.int32)], axis=1)

    @pl.when(i == 0)
    def _():
        bidx = lax.broadcasted_iota(jnp.int32, (NBLK, E), 0)
        po_i = jnp.broadcast_to(po.astype(jnp.int32), (NBLK, E))
        cnt = jnp.sum((bidx * BM >= po_i).astype(jnp.int32),
                      axis=1, keepdims=True)
        be_ref[...] = cnt - 1


def _route(rinfo, tri, ut8):
    return pl.pallas_call(
        _route_body,
        grid=(M // BM,),
        in_specs=[
            pl.BlockSpec((M, E), lambda i: (0, 0)),
            pl.BlockSpec((BM, E), lambda i: (i, 0)),
            pl.BlockSpec((BM, M), lambda i: (i, 0)),
            pl.BlockSpec((E, E), lambda i: (0, 0)),
        ],
        out_specs=[
            pl.BlockSpec((BM, 2), lambda i: (i, 0)),
            pl.BlockSpec((NBLK, 1), lambda i: (0, 0)),
        ],
        out_shape=[
            jax.ShapeDtypeStruct((M, 2), jnp.int32),
            jax.ShapeDtypeStruct((NBLK, 1), jnp.int32),
        ],
    )(rinfo, rinfo, tri, ut8)


def _sc_dispatch_body(h1_hbm, dst_hbm, out_hbm, idx_v, rows_v):
    wid = lax.axis_index("s") * 2 + lax.axis_index("c")
    pltpu.sync_copy(dst_hbm.at[wid], idx_v)
    pltpu.sync_copy(h1_hbm.at[pl.ds((wid % (M // PPW)) * PPW, PPW)], rows_v)
    pltpu.sync_copy(rows_v, out_hbm.at[idx_v])


def _sc_dispatch(h1, dst_w):
    fn = functools.partial(
        pl.kernel,
        mesh=plsc.VectorSubcoreMesh(core_axis_name="c", subcore_axis_name="s"),
        out_type=jax.ShapeDtypeStruct((CAP, H), jnp.float32),
        scratch_types=[
            pltpu.VMEM((PPW,), jnp.int32),
            pltpu.VMEM((PPW, H), jnp.float32),
        ],
    )(_sc_dispatch_body)
    return fn(h1, dst_w)


def _moe_body(be_ref, x_ref, w1_ref, b1_ref, w2_ref, b2_ref, y_ref):
    fb = pl.program_id(1)
    he = jnp.maximum(
        jnp.dot(x_ref[...], w1_ref[0], preferred_element_type=jnp.float32)
        + b1_ref[0], 0.0)
    part = jnp.dot(he, w2_ref[0], preferred_element_type=jnp.float32)

    @pl.when(fb == 0)
    def _():
        y_ref[...] = part + b2_ref[0]

    @pl.when(fb != 0)
    def _():
        y_ref[...] = y_ref[...] + part


def _moe(xg, be, w1, b1, w2, b2):
    return pl.pallas_call(
        _moe_body,
        grid_spec=pltpu.PrefetchScalarGridSpec(
            num_scalar_prefetch=1,
            grid=(NBLK, NFB),
            in_specs=[
                pl.BlockSpec((BM, H), lambda i, f, be: (i, 0)),
                pl.BlockSpec((1, H, BF), lambda i, f, be: (be[i], 0, f)),
                pl.BlockSpec((1, 1, BF), lambda i, f, be: (be[i], 0, f)),
                pl.BlockSpec((1, BF, H), lambda i, f, be: (be[i], f, 0)),
                pl.BlockSpec((1, 1, H), lambda i, f, be: (be[i], 0, 0)),
            ],
            out_specs=pl.BlockSpec((BM, H), lambda i, f, be: (i, 0)),
        ),
        out_shape=jax.ShapeDtypeStruct((CAP, H), jnp.float32),
    )(be, xg, w1, b1.reshape(E, 1, F), w2, b2.reshape(E, 1, H))


def _sc_combine_body(y_hbm, dst_hbm, out_hbm, idx_v, rows_v, sem):
    wid = lax.axis_index("s") * 2 + lax.axis_index("c")
    pltpu.sync_copy(dst_hbm.at[wid], idx_v)
    pltpu.async_copy(y_hbm.at[idx_v], rows_v, sem).wait()
    pltpu.sync_copy(rows_v, out_hbm.at[pl.ds(wid * PPW, PPW)])


def _sc_combine(y, dst_w):
    fn = functools.partial(
        pl.kernel,
        mesh=plsc.VectorSubcoreMesh(core_axis_name="c", subcore_axis_name="s"),
        out_type=jax.ShapeDtypeStruct((PAIRS, H), jnp.float32),
        scratch_types=[
            pltpu.VMEM((PPW,), jnp.int32),
            pltpu.VMEM((PPW, H), jnp.float32),
            pltpu.SemaphoreType.DMA,
        ],
    )(_sc_combine_body)
    return fn(y, dst_w)


def _final_body(h1_ref, ya_ref, yb_ref, r_ref, mom_ref, g_ref, b_ref,
                out_ref, mn_ref):
    r = r_ref[...]
    moe = r[:, 2:3] * ya_ref[...] + r[:, 3:4] * yb_ref[...]
    mnew = MU * mom_ref[...] + GAMMA * moe
    mn_ref[...] = mnew
    x = h1_ref[...] - mnew
    mu = jnp.mean(x, axis=1, keepdims=True)
    var = jnp.mean((x - mu) ** 2, axis=1, keepdims=True)
    out_ref[...] = (x - mu) / jnp.sqrt(var + 1e-5) * g_ref[...] + b_ref[...]


def _final(h1, ya, yb, rinfo, mom, g, b):
    return pl.pallas_call(
        _final_body,
        grid=(M // BM,),
        in_specs=[
            pl.BlockSpec((BM, H), lambda i: (i, 0)),
            pl.BlockSpec((BM, H), lambda i: (i, 0)),
            pl.BlockSpec((BM, H), lambda i: (i, 0)),
            pl.BlockSpec((BM, E), lambda i: (i, 0)),
            pl.BlockSpec((BM, H), lambda i: (i, 0)),
            pl.BlockSpec((1, H), lambda i: (0, 0)),
            pl.BlockSpec((1, H), lambda i: (0, 0)),
        ],
        out_specs=[
            pl.BlockSpec((BM, H), lambda i: (i, 0)),
            pl.BlockSpec((BM, H), lambda i: (i, 0)),
        ],
        out_shape=[
            jax.ShapeDtypeStruct((M, H), jnp.float32),
            jax.ShapeDtypeStruct((M, H), jnp.float32),
        ],
    )(h1, ya, yb, rinfo, mom, g, b)


def kernel(h, h_cache, pos_encoding, momentum, Wq, Wk, Wv, Wo,
           ln1_g, ln1_b, ln2_g, ln2_b, Wg, W1, b1, W2, b2):
    h2d = h.reshape(M, H)
    h_all = jnp.concatenate([h_cache.reshape(L, H), h2d], axis=0)

    (qh,) = _proj_heads(h2d, Wq)
    kh, vh = _proj_heads(h_all, jnp.concatenate([Wk, Wv], axis=1))
    pos_pad = jnp.pad(pos_encoding, ((0, 0), (BA, BA)))

    att = _flash(qh, kh, vh, pos_pad)

    h1, rinfo = _post_attn(att, h2d, Wo, ln1_g.reshape(1, H),
                           ln1_b.reshape(1, H), Wg)

    tri = (lax.broadcasted_iota(jnp.int32, (M, M), 1)
           < lax.broadcasted_iota(jnp.int32, (M, M), 0)).astype(jnp.float32)
    ut8 = (lax.broadcasted_iota(jnp.int32, (E, E), 0)
           < lax.broadcasted_iota(jnp.int32, (E, E), 1)).astype(jnp.float32)
    dst, be = _route(rinfo, tri, ut8)

    dst_w = dst.T.reshape(NW, PPW)
    be1d = be.reshape(NBLK)

    xg = _sc_dispatch(h1, dst_w)
    y = _moe(xg, be1d, W1, b1, W2, b2)
    yab = _sc_combine(y, dst_w)

    h_out, mnew = _final(h1, yab[:M], yab[M:], rinfo, momentum.reshape(M, H),
                         ln2_g.reshape(1, H), ln2_b.reshape(1, H))
    return h_out.reshape(B, M, H), mnew.reshape(B, M, H)


# speedup vs baseline: 1.0012x; 1.0012x over previous
import functools
import math

import jax
import jax.numpy as jnp
from jax import lax
from jax.experimental import pallas as pl
from jax.experimental.pallas import tpu as pltpu
from jax.experimental.pallas import tpu_sc as plsc

B, M, L, H, NH, D, E, TOPK, F = 1, 2048, 2048, 768, 12, 64, 8, 2, 3072
MU, GAMMA = 0.7, 1.0

BM = 256
BA = 1024
KB = L // BA + 1
CAP = 2 * M + E * BM
NBLK = CAP // BM
BF = 3072
NFB = F // BF
NW = 32
PAIRS = 2 * M
PPW = PAIRS // NW
SCALE = 1.0 / math.sqrt(float(H))


def _proj_body(x_ref, w_ref, *o_refs):
    r = jnp.dot(x_ref[...], w_ref[...], preferred_element_type=jnp.float32)
    for g, o_ref in enumerate(o_refs):
        for h in range(NH):
            o_ref[h] = r[:, (g * NH + h) * D:(g * NH + h + 1) * D]


def _proj_heads(x, w):
    n, k = x.shape
    k2, m = w.shape
    ng = m // H
    return pl.pallas_call(
        _proj_body,
        grid=(n // BM,),
        in_specs=[
            pl.BlockSpec((BM, k), lambda i: (i, 0)),
            pl.BlockSpec((k2, m), lambda i: (0, 0)),
        ],
        out_specs=[pl.BlockSpec((NH, BM, D), lambda i: (0, i, 0))] * ng,
        out_shape=[jax.ShapeDtypeStruct((NH, n, D), jnp.float32)] * ng,
    )(x, w)


def _flash_body(q_ref, k_ref, v_ref, pos_ref, o_ref, s_s, v_s, pz_s):
    kb = pl.program_id(2)
    q = q_ref[0]
    k = k_ref[0]
    s = lax.dot_general(q, k, (((1,), (1,)), ((), ())),
                        preferred_element_type=jnp.float32)
    ri = lax.broadcasted_iota(jnp.int32, (BA, BA), 0)
    cj = lax.broadcasted_iota(jnp.int32, (BA, BA), 1)

    @pl.when(kb == 0)
    def _():
        pz_s[...] = jnp.zeros_like(pz_s)

    pzb = jnp.dot(q, pos_ref[:, pl.ds((kb + 1) * BA, BA)],
                  preferred_element_type=jnp.float32)
    rotb = pltpu.roll(pzb, 0, 1, stride=1, stride_axis=0)
    pbias = jnp.where(cj >= ri, rotb, pz_s[...])
    pz_s[...] = rotb
    s = (s + pbias) * SCALE

    @pl.when((kb == 0) | (kb == KB - 1))
    def _():
        rel = kb * BA + cj - ri
        valid = (rel >= 0) & (rel < L)
        s_s[:, pl.ds(kb * BA, BA)] = jnp.where(valid, s, -1e30)

    @pl.when((kb != 0) & (kb != KB - 1))
    def _():
        s_s[:, pl.ds(kb * BA, BA)] = s

    v_s[pl.ds(kb * BA, BA), :] = v_ref[0]

    @pl.when(kb == KB - 1)
    def _():
        sall = s_s[...]
        mrow = jnp.max(sall, axis=1, keepdims=True)
        pm = jnp.exp(sall - mrow)
        p = pm / jnp.sum(pm, axis=1, keepdims=True)
        o_ref[0] = jnp.dot(p, v_s[...], preferred_element_type=jnp.float32)


def _flash(qh, kh, vh, pos_pad):
    return pl.pallas_call(
        _flash_body,
        grid=(NH, M // BA, KB),
        in_specs=[
            pl.BlockSpec((1, BA, D), lambda h, i, j: (h, i, 0)),
            pl.BlockSpec((1, BA, D), lambda h, i, j: (h, i + j, 0)),
            pl.BlockSpec((1, BA, D), lambda h, i, j: (h, i + j, 0)),
            pl.BlockSpec((D, L + 2 * BA), lambda h, i, j: (0, 0)),
        ],
        out_specs=pl.BlockSpec((1, BA, D), lambda h, i, j: (h, i, 0)),
        out_shape=jax.ShapeDtypeStruct((NH, M, D), jnp.float32),
        scratch_shapes=[
            pltpu.VMEM((BA, KB * BA), jnp.float32),
            pltpu.VMEM((KB * BA, D), jnp.float32),
            pltpu.VMEM((BA, BA), jnp.float32),
        ],
    )(qh, kh, vh, pos_pad)


def _post_body(a_ref, h_ref, wo_ref, g_ref, b_ref, wg_ref, h1_ref, r_ref):
    att = jnp.concatenate([a_ref[h] for h in range(NH)], axis=1)
    x = jnp.dot(att, wo_ref[...],
                preferred_element_type=jnp.float32) + h_ref[...]
    mu = jnp.mean(x, axis=1, keepdims=True)
    var = jnp.mean((x - mu) ** 2, axis=1, keepdims=True)
    xn = (x - mu) / jnp.sqrt(var + 1e-5) * g_ref[...] + b_ref[...]
    h1_ref[...] = xn

    logits = jnp.dot(xn, wg_ref[...], preferred_element_type=jnp.float32)
    eidx = lax.broadcasted_iota(jnp.int32, (BM, E), 1)
    v1 = jnp.max(logits, axis=1, keepdims=True)
    i1 = jnp.min(jnp.where(logits == v1, eidx, E), axis=1, keepdims=True)
    l2 = jnp.where(eidx == i1, -jnp.inf, logits)
    v2 = jnp.max(l2, axis=1, keepdims=True)
    i2 = jnp.min(jnp.where(l2 == v2, eidx, E), axis=1, keepdims=True)
    e2 = jnp.exp(v2 - v1)
    w1 = 1.0 / (1.0 + e2)
    w2 = e2 / (1.0 + e2)
    r = jnp.where(eidx == 0, i1.astype(jnp.float32), 0.0)
    r = r + jnp.where(eidx == 1, i2.astype(jnp.float32), 0.0)
    r = r + jnp.where(eidx == 2, w1, 0.0)
    r = r + jnp.where(eidx == 3, w2, 0.0)
    r_ref[...] = r


def _post_attn(att, h2d, wo, g, b, wg):
    return pl.pallas_call(
        _post_body,
        grid=(M // BM,),
        in_specs=[
            pl.BlockSpec((NH, BM, D), lambda i: (0, i, 0)),
            pl.BlockSpec((BM, H), lambda i: (i, 0)),
            pl.BlockSpec((H, H), lambda i: (0, 0)),
            pl.BlockSpec((1, H), lambda i: (0, 0)),
            pl.BlockSpec((1, H), lambda i: (0, 0)),
            pl.BlockSpec((H, E), lambda i: (0, 0)),
        ],
        out_specs=[
            pl.BlockSpec((BM, H), lambda i: (i, 0)),
            pl.BlockSpec((BM, E), lambda i: (i, 0)),
        ],
        out_shape=[
            jax.ShapeDtypeStruct((M, H), jnp.float32),
            jax.ShapeDtypeStruct((M, E), jnp.float32),
        ],
    )(att, h2d, wo, g, b, wg)


def _route_body(rfull_ref, rchunk_ref, tri_ref, ut_ref, dst_ref, be_ref):
    i = pl.program_id(0)
    rfull = rfull_ref[...]
    eidx_f = lax.broadcasted_iota(jnp.int32, (M, E), 1).astype(jnp.float32)
    oh1 = (eidx_f == rfull[:, 0:1]).astype(jnp.float32)
    oh2 = (eidx_f == rfull[:, 1:2]).astype(jnp.float32)
    counts1 = jnp.sum(oh1, axis=0, keepdims=True)
    counts = counts1 + jnp.sum(oh2, axis=0, keepdims=True)
    counts_i = counts.astype(jnp.int32)
    pc = ((counts_i + BM - 1) // BM) * BM
    pcf = jnp.broadcast_to(pc.astype(jnp.float32), (E, E))
    po8 = jnp.dot(pcf, ut_ref[...], preferred_element_type=jnp.float32)
    po = po8[0:1, :]

    tri = tri_ref[...]
    excl1 = jnp.dot(tri, oh1, preferred_element_type=jnp.float32)
    excl2 = jnp.dot(tri, oh2, preferred_element_type=jnp.float32)

    rchunk = rchunk_ref[...]
    eidx_c = lax.broadcasted_iota(jnp.int32, (BM, E), 1).astype(jnp.float32)
    oh1c = (eidx_c == rchunk[:, 0:1]).astype(jnp.float32)
    oh2c = (eidx_c == rchunk[:, 1:2]).astype(jnp.float32)
    rank1 = jnp.sum(excl1 * oh1c, axis=1, keepdims=True)
    rank2 = jnp.sum(excl2 * oh2c, axis=1, keepdims=True) + \
        jnp.sum(oh1c * 0.0 + oh2c * counts1, axis=1, keepdims=True)
    dst1 = jnp.sum(oh1c * po, axis=1, keepdims=True) + rank1
    dst2 = jnp.sum(oh2c * po, axis=1, keepdims=True) + rank2
    dst_ref[...] = jnp.concatenate(
        [dst1.astype(jnp.int32), dst2.astype(jnp.int32)], axis=1)

    @pl.when(i == 0)
    def _():
        bidx = lax.broadcasted_iota(jnp.int32, (NBLK, E), 0)
        po_i = jnp.broadcast_to(po.astype(jnp.int32), (NBLK, E))
        cnt = jnp.sum((bidx * BM >= po_i).astype(jnp.int32),
                      axis=1, keepdims=True)
        be_ref[...] = cnt - 1


def _route(rinfo, tri, ut8):
    return pl.pallas_call(
        _route_body,
        grid=(M // BM,),
        in_specs=[
            pl.BlockSpec((M, E), lambda i: (0, 0)),
            pl.BlockSpec((BM, E), lambda i: (i, 0)),
            pl.BlockSpec((BM, M), lambda i: (i, 0)),
            pl.BlockSpec((E, E), lambda i: (0, 0)),
        ],
        out_specs=[
            pl.BlockSpec((BM, 2), lambda i: (i, 0)),
            pl.BlockSpec((NBLK, 1), lambda i: (0, 0)),
        ],
        out_shape=[
            jax.ShapeDtypeStruct((M, 2), jnp.int32),
            jax.ShapeDtypeStruct((NBLK, 1), jnp.int32),
        ],
    )(rinfo, rinfo, tri, ut8)


def _sc_dispatch_body(h1_hbm, dst_hbm, out_hbm, idx_v, rows_v):
    wid = lax.axis_index("s") * 2 + lax.axis_index("c")
    pltpu.sync_copy(dst_hbm.at[wid], idx_v)
    pltpu.sync_copy(h1_hbm.at[pl.ds((wid % (M // PPW)) * PPW, PPW)], rows_v)
    pltpu.sync_copy(rows_v, out_hbm.at[idx_v])


def _sc_dispatch(h1, dst_w):
    fn = functools.partial(
        pl.kernel,
        mesh=plsc.VectorSubcoreMesh(core_axis_name="c", subcore_axis_name="s"),
        out_type=jax.ShapeDtypeStruct((CAP, H), jnp.float32),
        scratch_types=[
            pltpu.VMEM((PPW,), jnp.int32),
            pltpu.VMEM((PPW, H), jnp.float32),
        ],
    )(_sc_dispatch_body)
    return fn(h1, dst_w)


def _moe_body(be_ref, x_ref, w1_ref, b1_ref, w2_ref, b2_ref, y_ref):
    fb = pl.program_id(1)
    he = jnp.maximum(
        jnp.dot(x_ref[...], w1_ref[0], preferred_element_type=jnp.float32)
        + b1_ref[0], 0.0)
    part = jnp.dot(he, w2_ref[0], preferred_element_type=jnp.float32)

    @pl.when(fb == 0)
    def _():
        y_ref[...] = part + b2_ref[0]

    @pl.when(fb != 0)
    def _():
        y_ref[...] = y_ref[...] + part


def _moe(xg, be, w1, b1, w2, b2):
    return pl.pallas_call(
        _moe_body,
        grid_spec=pltpu.PrefetchScalarGridSpec(
            num_scalar_prefetch=1,
            grid=(NBLK, NFB),
            in_specs=[
                pl.BlockSpec((BM, H), lambda i, f, be: (i, 0)),
                pl.BlockSpec((1, H, BF), lambda i, f, be: (be[i], 0, f)),
                pl.BlockSpec((1, 1, BF), lambda i, f, be: (be[i], 0, f)),
                pl.BlockSpec((1, BF, H), lambda i, f, be: (be[i], f, 0)),
                pl.BlockSpec((1, 1, H), lambda i, f, be: (be[i], 0, 0)),
            ],
            out_specs=pl.BlockSpec((BM, H), lambda i, f, be: (i, 0)),
        ),
        out_shape=jax.ShapeDtypeStruct((CAP, H), jnp.float32),
    )(be, xg, w1, b1.reshape(E, 1, F), w2, b2.reshape(E, 1, H))


def _sc_combine_body(y_hbm, dst_hbm, out_hbm, idx_v, rows_v, sem):
    wid = lax.axis_index("s") * 2 + lax.axis_index("c")
    pltpu.sync_copy(dst_hbm.at[wid], idx_v)
    pltpu.async_copy(y_hbm.at[idx_v], rows_v, sem).wait()
    pltpu.sync_copy(rows_v, out_hbm.at[pl.ds(wid * PPW, PPW)])


def _sc_combine(y, dst_w):
    fn = functools.partial(
        pl.kernel,
        mesh=plsc.VectorSubcoreMesh(core_axis_name="c", subcore_axis_name="s"),
        out_type=jax.ShapeDtypeStruct((PAIRS, H), jnp.float32),
        scratch_types=[
            pltpu.VMEM((PPW,), jnp.int32),
            pltpu.VMEM((PPW, H), jnp.float32),
            pltpu.SemaphoreType.DMA,
        ],
    )(_sc_combine_body)
    return fn(y, dst_w)


def _final_body(h1_ref, ya_ref, yb_ref, r_ref, mom_ref, g_ref, b_ref,
                out_ref, mn_ref):
    r = r_ref[...]
    moe = r[:, 2:3] * ya_ref[...] + r[:, 3:4] * yb_ref[...]
    mnew = MU * mom_ref[...] + GAMMA * moe
    mn_ref[...] = mnew
    x = h1_ref[...] - mnew
    mu = jnp.mean(x, axis=1, keepdims=True)
    var = jnp.mean((x - mu) ** 2, axis=1, keepdims=True)
    out_ref[...] = (x - mu) / jnp.sqrt(var + 1e-5) * g_ref[...] + b_ref[...]


def _final(h1, ya, yb, rinfo, mom, g, b):
    return pl.pallas_call(
        _final_body,
        grid=(M // BM,),
        in_specs=[
            pl.BlockSpec((BM, H), lambda i: (i, 0)),
            pl.BlockSpec((BM, H), lambda i: (i, 0)),
            pl.BlockSpec((BM, H), lambda i: (i, 0)),
            pl.BlockSpec((BM, E), lambda i: (i, 0)),
            pl.BlockSpec((BM, H), lambda i: (i, 0)),
            pl.BlockSpec((1, H), lambda i: (0, 0)),
            pl.BlockSpec((1, H), lambda i: (0, 0)),
        ],
        out_specs=[
            pl.BlockSpec((BM, H), lambda i: (i, 0)),
            pl.BlockSpec((BM, H), lambda i: (i, 0)),
        ],
        out_shape=[
            jax.ShapeDtypeStruct((M, H), jnp.float32),
            jax.ShapeDtypeStruct((M, H), jnp.float32),
        ],
    )(h1, ya, yb, rinfo, mom, g, b)


def kernel(h, h_cache, pos_encoding, momentum, Wq, Wk, Wv, Wo,
           ln1_g, ln1_b, ln2_g, ln2_b, Wg, W1, b1, W2, b2):
    h2d = h.reshape(M, H)
    h_all = jnp.concatenate([h_cache.reshape(L, H), h2d], axis=0)

    (qh,) = _proj_heads(h2d, Wq)
    kh, vh = _proj_heads(h_all, jnp.concatenate([Wk, Wv], axis=1))
    pos_pad = jnp.pad(pos_encoding, ((0, 0), (BA, BA)))

    att = _flash(qh, kh, vh, pos_pad)

    h1, rinfo = _post_attn(att, h2d, Wo, ln1_g.reshape(1, H),
                           ln1_b.reshape(1, H), Wg)

    tri = (lax.broadcasted_iota(jnp.int32, (M, M), 1)
           < lax.broadcasted_iota(jnp.int32, (M, M), 0)).astype(jnp.float32)
    ut8 = (lax.broadcasted_iota(jnp.int32, (E, E), 0)
           < lax.broadcasted_iota(jnp.int32, (E, E), 1)).astype(jnp.float32)
    dst, be = _route(rinfo, tri, ut8)

    dst_w = dst.T.reshape(NW, PPW)
    be1d = be.reshape(NBLK)

    xg = _sc_dispatch(h1, dst_w)
    y = _moe(xg, be1d, W1, b1, W2, b2)
    yab = _sc_combine(y, dst_w)

    h_out, mnew = _final(h1, yab[:M], yab[M:], rinfo, momentum.reshape(M, H),
                         ln2_g.reshape(1, H), ln2_b.reshape(1, H))
    return h_out.reshape(B, M, H), mnew.reshape(B, M, H)
